# Initial kernel scaffold; baseline (speedup 1.0000x reference)
import jax
import jax.numpy as jnp
from jax.experimental import pallas as pl


def kernel(query, document, query_len, W1, b1, W2, b2, W3, b3, Wg, bg):
    raise NotImplementedError("write your pallas kernel here")


# speedup vs baseline: 11.9624x; 11.9624x over previous
import jax
import jax.numpy as jnp
from jax.experimental import pallas as pl
from jax.experimental.pallas import tpu as pltpu

_B, _LQ, _LD, _D, _NBINS = 32, 20, 4096, 300, 30


def _drmm_body(ql_ref, sc_ref, q_ref, d_ref, w1_ref, b1_ref, w2_ref, wg_ref,
               out_ref):
    b = pl.program_id(0)

    q = q_ref[0]
    qn = q * jax.lax.rsqrt(jnp.sum(q * q, axis=1, keepdims=True))
    d = d_ref[0]
    dinv = jax.lax.rsqrt(jnp.sum(d * d, axis=1))

    s = jax.lax.dot_general(qn, d, (((1,), (1,)), ((), ())),
                            preferred_element_type=jnp.float32)
    x = s * dinv[None, :]

    width = jnp.float32(2.0 / _NBINS)
    idx = jnp.floor((x + 1.0) / width)
    idx = jnp.clip(idx, 0.0, _NBINS - 1)
    idxv = jnp.where((x >= -1.0) & (x <= 1.0), idx, jnp.float32(-1.0))

    lane = jax.lax.broadcasted_iota(jnp.int32, (1, _NBINS), 1)
    h = jnp.zeros((_LQ, _NBINS), jnp.float32)
    for j in range(_NBINS):
        cnt = jnp.sum(jnp.where(idxv == j, 1.0, 0.0), axis=1,
                      keepdims=True)
        h = h + cnt * (lane == j).astype(jnp.float32)

    ql = ql_ref[b]
    row = jax.lax.broadcasted_iota(jnp.int32, (_LQ, 1), 0)
    h = h * (row < ql).astype(jnp.float32)
    h = jnp.log1p(h)

    z = jnp.tanh(jax.lax.dot_general(h, w1_ref[...], (((1,), (1,)), ((), ())),
                                     preferred_element_type=jnp.float32)
                 + b1_ref[...])
    z = jnp.tanh(jnp.sum(z * w2_ref[...], axis=1, keepdims=True) + sc_ref[0])
    z = jnp.tanh(z * sc_ref[2] + sc_ref[1])

    g = jnp.sum(qn * wg_ref[...], axis=1, keepdims=True) + sc_ref[3]
    g = g - jnp.max(g)
    e = jnp.exp(g)
    g = e / jnp.sum(e)

    out_ref[...] = jnp.sum(z * g, axis=(0, 1), keepdims=True).reshape(1, 1, 1)


def kernel(query, document, query_len, W1, b1, W2, b2, W3, b3, Wg, bg):
    scalars = jnp.concatenate([b2.reshape(1), b3.reshape(1),
                               W3.reshape(1), bg.reshape(1)])
    grid_spec = pltpu.PrefetchScalarGridSpec(
        num_scalar_prefetch=2,
        grid=(_B,),
        in_specs=[
            pl.BlockSpec((1, _LQ, _D), lambda b, ql, sc: (b, 0, 0)),
            pl.BlockSpec((1, _LD, _D), lambda b, ql, sc: (b, 0, 0)),
            pl.BlockSpec((5, _NBINS), lambda b, ql, sc: (0, 0)),
            pl.BlockSpec((1, 5), lambda b, ql, sc: (0, 0)),
            pl.BlockSpec((1, 5), lambda b, ql, sc: (0, 0)),
            pl.BlockSpec((1, _D), lambda b, ql, sc: (0, 0)),
        ],
        out_specs=pl.BlockSpec((1, 1, 1), lambda b, ql, sc: (b, 0, 0)),
    )
    out = pl.pallas_call(
        _drmm_body,
        grid_spec=grid_spec,
        out_shape=jax.ShapeDtypeStruct((_B, 1, 1), jnp.float32),
        compiler_params=pltpu.CompilerParams(
            dimension_semantics=("arbitrary",),
        ),
    )(query_len, scalars, query, document, W1, b1.reshape(1, 5), W2, Wg)
    return out[:, 0, 0]
